# Initial kernel scaffold; baseline (speedup 1.0000x reference)
import jax
import jax.numpy as jnp
from jax.experimental import pallas as pl


def kernel(idx_f0, W_f0, idx_f1, W_f1, idx_f2, W_f2, idx_f3, W_f3, idx_f4, W_f4, idx_f5, W_f5, idx_f6, W_f6, idx_f7, W_f7, idx_f8, W_f8, idx_f9, W_f9, idx_f10, W_f10, idx_f11, W_f11, idx_f12, W_f12, idx_f13, W_f13, idx_f14, W_f14, idx_f15, W_f15, idx_f16, W_f16, idx_f17, W_f17, idx_f18, W_f18, idx_f19, W_f19, idx_f20, W_f20, idx_f21, W_f21, idx_f22, W_f22, idx_f23, W_f23, idx_f24, W_f24, idx_f25, W_f25):
    raise NotImplementedError("write your pallas kernel here")


# speedup vs baseline: 3.5165x; 3.5165x over previous
import functools

import jax
import jax.numpy as jnp
import numpy as np
from jax import lax
from jax.experimental import pallas as pl
from jax.experimental.pallas import tpu as pltpu
from jax.experimental.pallas import tpu_sc as plsc

NUM_FIELDS = 26
BATCH = 16384
VOCAB = 1000
EMB_DIM = 128
SCALE = float(1.0 / np.sqrt(float(NUM_FIELDS)))

NC = 2
NS = 16
NW = NC * NS
B_PER_W = BATCH // NW
CHUNK = 128
NCHUNK = B_PER_W // CHUNK
IDX_ROWS = BATCH // CHUNK


def _sc_combine(W_all, idx2d):
    mesh = plsc.VectorSubcoreMesh(core_axis_name="c", subcore_axis_name="s")

    @functools.partial(
        pl.kernel,
        mesh=mesh,
        out_type=jax.ShapeDtypeStruct((BATCH, EMB_DIM), jnp.float32),
        scratch_types=[
            pltpu.VMEM((NCHUNK, CHUNK), jnp.int32),
            pltpu.VMEM((NCHUNK, CHUNK), jnp.int32),
            pltpu.VMEM((CHUNK, EMB_DIM), jnp.float32),
            pltpu.VMEM((B_PER_W, EMB_DIM), jnp.float32),
            pltpu.SemaphoreType.DMA,
        ],
    )
    def body(W_hbm, idx_hbm, out_hbm, idx_v, idxo_v, rows_v, acc_v, sem):
        wid = lax.axis_index("s") * NC + lax.axis_index("c")
        base = wid * B_PER_W
        zero = jnp.zeros((16,), jnp.float32)

        def zrow(r, _):
            for j in range(EMB_DIM // 16):
                acc_v[r, pl.ds(j * 16, 16)] = zero
            return 0

        lax.fori_loop(0, B_PER_W, zrow, 0, unroll=4)

        def field_body(f, _):
            row0 = f * IDX_ROWS + wid * NCHUNK
            pltpu.sync_copy(idx_hbm.at[pl.ds(row0, NCHUNK)], idx_v)
            off = f * VOCAB

            def offrow(c, _):
                for j in range(CHUNK // 16):
                    sl = pl.ds(j * 16, 16)
                    idxo_v[c, sl] = idx_v[c, sl] + off
                return 0

            lax.fori_loop(0, NCHUNK, offrow, 0)

            for c in range(NCHUNK):
                pltpu.async_copy(W_hbm.at[idxo_v.at[c]], rows_v, sem).wait()

                def accrow(r, _):
                    for j in range(EMB_DIM // 16):
                        sl = pl.ds(j * 16, 16)
                        acc_v[c * CHUNK + r, sl] = (
                            acc_v[c * CHUNK + r, sl] + rows_v[r, sl]
                        )
                    return 0

                lax.fori_loop(0, CHUNK, accrow, 0, unroll=2)
            return 0

        lax.fori_loop(0, NUM_FIELDS, field_body, 0)

        for c in range(NCHUNK):

            def scrow(r, _):
                for j in range(EMB_DIM // 16):
                    sl = pl.ds(j * 16, 16)
                    rows_v[r, sl] = acc_v[c * CHUNK + r, sl] * SCALE
                return 0

            lax.fori_loop(0, CHUNK, scrow, 0, unroll=2)
            pltpu.sync_copy(rows_v, out_hbm.at[pl.ds(base + c * CHUNK, CHUNK)])

    return body(W_all, idx2d)


def kernel(idx_f0, W_f0, idx_f1, W_f1, idx_f2, W_f2, idx_f3, W_f3, idx_f4, W_f4, idx_f5, W_f5, idx_f6, W_f6, idx_f7, W_f7, idx_f8, W_f8, idx_f9, W_f9, idx_f10, W_f10, idx_f11, W_f11, idx_f12, W_f12, idx_f13, W_f13, idx_f14, W_f14, idx_f15, W_f15, idx_f16, W_f16, idx_f17, W_f17, idx_f18, W_f18, idx_f19, W_f19, idx_f20, W_f20, idx_f21, W_f21, idx_f22, W_f22, idx_f23, W_f23, idx_f24, W_f24, idx_f25, W_f25):
    fields = locals()
    Ws = [fields[f"W_f{i}"] for i in range(NUM_FIELDS)]
    idxs = [fields[f"idx_f{i}"] for i in range(NUM_FIELDS)]
    W_all = jnp.concatenate(Ws, axis=0)
    idx2d = (
        jnp.stack(idxs, axis=0)
        .astype(jnp.int32)
        .reshape(NUM_FIELDS * BATCH // CHUNK, CHUNK)
    )
    return _sc_combine(W_all, idx2d)


# speedup vs baseline: 7.4843x; 2.1283x over previous
import functools

import jax
import jax.numpy as jnp
import numpy as np
from jax import lax
from jax.experimental import pallas as pl
from jax.experimental.pallas import tpu as pltpu
from jax.experimental.pallas import tpu_sc as plsc

NUM_FIELDS = 26
BATCH = 16384
VOCAB = 1000
EMB_DIM = 128
SCALE = float(1.0 / np.sqrt(float(NUM_FIELDS)))

NC = 2
NS = 16
NW = NC * NS
B_PER_W = BATCH // NW
CHUNK = 128
NCHUNK = B_PER_W // CHUNK
NT = NUM_FIELDS * NCHUNK
NLANE = EMB_DIM // 16


def _sc_combine(W_all, idx3d):
    mesh = plsc.VectorSubcoreMesh(core_axis_name="c", subcore_axis_name="s")

    @functools.partial(
        pl.kernel,
        mesh=mesh,
        out_type=jax.ShapeDtypeStruct((BATCH, EMB_DIM), jnp.float32),
        scratch_types=[
            pltpu.VMEM((NT, CHUNK), jnp.int32),
            pltpu.VMEM((CHUNK, EMB_DIM), jnp.float32),
            pltpu.VMEM((CHUNK, EMB_DIM), jnp.float32),
            pltpu.VMEM((B_PER_W, EMB_DIM), jnp.float32),
            pltpu.SemaphoreType.DMA,
            pltpu.SemaphoreType.DMA,
        ],
    )
    def body(W_hbm, idx_hbm, out_hbm, idx_v, rows0, rows1, acc_v, sem0, sem1):
        wid = lax.axis_index("s") * NC + lax.axis_index("c")
        base = wid * B_PER_W
        rows = (rows0, rows1)
        sems = (sem0, sem1)
        zero = jnp.zeros((16,), jnp.float32)

        pltpu.sync_copy(idx_hbm.at[wid], idx_v)

        def offbody(f, _):
            off = f * VOCAB
            for c in range(NCHUNK):
                row = c * NUM_FIELDS + f
                for j in range(CHUNK // 16):
                    sl = pl.ds(j * 16, 16)
                    idx_v[row, sl] = idx_v[row, sl] + off
            return 0

        lax.fori_loop(0, NUM_FIELDS, offbody, 0)

        def zrow(r, _):
            for j in range(NLANE):
                acc_v[r, pl.ds(j * 16, 16)] = zero
            return 0

        lax.fori_loop(0, B_PER_W, zrow, 0, unroll=4)

        def gstart(t, b):
            pltpu.make_async_copy(W_hbm.at[idx_v.at[t]], rows[b], sems[b]).start()

        def gwait(t, b):
            pltpu.make_async_copy(W_hbm.at[idx_v.at[t]], rows[b], sems[b]).wait()

        for c in range(NCHUNK):
            cbase = c * CHUNK
            tbase = c * NUM_FIELDS
            gstart(tbase, 0)

            def fbody(ff, _):
                for b in range(2):
                    f = ff * 2 + b
                    t = tbase + f

                    @pl.when(f + 1 < NUM_FIELDS)
                    def _():
                        gstart(t + 1, 1 - b)

                    gwait(t, b)
                    buf = rows[b]

                    def accrow(r, _):
                        for j in range(NLANE):
                            sl = pl.ds(j * 16, 16)
                            plsc.addupdate(acc_v.at[cbase + r, sl], buf[r, sl])
                        return 0

                    lax.fori_loop(0, CHUNK, accrow, 0, unroll=2)
                return 0

            lax.fori_loop(0, NUM_FIELDS // 2, fbody, 0)

        for c in range(NCHUNK):
            cbase = c * CHUNK

            def scrow(r, _):
                for j in range(NLANE):
                    sl = pl.ds(j * 16, 16)
                    rows0[r, sl] = acc_v[cbase + r, sl] * SCALE
                return 0

            lax.fori_loop(0, CHUNK, scrow, 0, unroll=2)
            pltpu.sync_copy(rows0, out_hbm.at[pl.ds(base + cbase, CHUNK)])

    return body(W_all, idx3d)


def kernel(idx_f0, W_f0, idx_f1, W_f1, idx_f2, W_f2, idx_f3, W_f3, idx_f4, W_f4, idx_f5, W_f5, idx_f6, W_f6, idx_f7, W_f7, idx_f8, W_f8, idx_f9, W_f9, idx_f10, W_f10, idx_f11, W_f11, idx_f12, W_f12, idx_f13, W_f13, idx_f14, W_f14, idx_f15, W_f15, idx_f16, W_f16, idx_f17, W_f17, idx_f18, W_f18, idx_f19, W_f19, idx_f20, W_f20, idx_f21, W_f21, idx_f22, W_f22, idx_f23, W_f23, idx_f24, W_f24, idx_f25, W_f25):
    fields = locals()
    Ws = [fields[f"W_f{i}"] for i in range(NUM_FIELDS)]
    idxs = [fields[f"idx_f{i}"] for i in range(NUM_FIELDS)]
    W_all = jnp.concatenate(Ws, axis=0)
    idx3d = (
        jnp.stack(idxs, axis=0)
        .astype(jnp.int32)
        .reshape(NUM_FIELDS, NW, NCHUNK, CHUNK)
        .transpose(1, 2, 0, 3)
        .reshape(NW, NT, CHUNK)
    )
    return _sc_combine(W_all, idx3d)
